# baseline (speedup 1.0000x reference)
import functools

import jax
import jax.numpy as jnp
from jax import lax
from jax.experimental import pallas as pl
from jax.experimental.pallas import tpu as pltpu
from jax.experimental.pallas import tpu_sc as plsc

N = 10000
D = 128
DE = 16
E = 320000
DO = D + DE

NC = 2
NS = 16
DH = D // NC
EPT = E // NS
B = 80
NB = EPT // B
NP = 10240
RPT = NP // NS
LAST = N - (NS - 1) * RPT
NBUF = 5


def _split(feat, eps):
    R = 2000

    def body(eps_ref, feat_ref, l_ref, r_ref, ls_ref, rs_ref):
        scale = 1.0 + eps_ref[0]
        l = feat_ref[:, :DH]
        r = feat_ref[:, DH:]
        l_ref[...] = l
        r_ref[...] = r
        ls_ref[...] = scale * l
        rs_ref[...] = scale * r

    return pl.pallas_call(
        body,
        grid=(N // R,),
        in_specs=[
            pl.BlockSpec(memory_space=pltpu.SMEM),
            pl.BlockSpec((R, D), lambda i: (i, 0)),
        ],
        out_specs=[pl.BlockSpec((R, DH), lambda i: (i, 0))] * 4,
        out_shape=[jax.ShapeDtypeStruct((N, DH), jnp.float32)] * 4,
    )(eps, feat)


def _sc_gin(featL, featR, featLs, featRs, src2, dst2, edge_w, zw):
    mesh = plsc.VectorSubcoreMesh(
        core_axis_name="c", subcore_axis_name="s", num_cores=NC,
        num_subcores=NS)

    @functools.partial(
        pl.kernel,
        out_type=jax.ShapeDtypeStruct((N, DO), jnp.float32),
        mesh=mesh,
        compiler_params=pltpu.CompilerParams(use_tc_tiling_on_sc=False),
        scratch_types=[
            pltpu.VMEM((NB, B), jnp.int32),
            pltpu.VMEM((NB, B), jnp.int32),
            pltpu.VMEM((NBUF, B, DH), jnp.float32),
            pltpu.VMEM((NBUF, B, DE), jnp.float32),
            pltpu.VMEM_SHARED((NP, DH), jnp.float32),
            pltpu.VMEM_SHARED((NP, DE), jnp.float32),
        ] + [pltpu.SemaphoreType.DMA] * (2 * NBUF),
    )
    def k(fL_hbm, fR_hbm, fLs_hbm, fRs_hbm, src_hbm, dst_hbm, ew_hbm,
          zw_hbm, out_hbm, src_v, dst_v, rows_v, ew_v, accf, accw, *sems):
        fsem = sems[:NBUF]
        wsem = sems[NBUF:]
        cid = lax.axis_index("c")
        sid = lax.axis_index("s")
        row0 = sid * RPT

        pltpu.sync_copy(src_hbm.at[sid], src_v)
        pltpu.sync_copy(dst_hbm.at[sid], dst_v)

        def seed(fs_hbm, rows):
            pltpu.sync_copy(fs_hbm.at[pl.ds(row0, rows)],
                            accf.at[pl.ds(row0, rows)])

        @pl.when(jnp.logical_and(cid == 0, sid < NS - 1))
        def _():
            seed(fLs_hbm, RPT)

        @pl.when(jnp.logical_and(cid == 0, sid == NS - 1))
        def _():
            seed(fLs_hbm, LAST)

        @pl.when(jnp.logical_and(cid == 1, sid < NS - 1))
        def _():
            seed(fRs_hbm, RPT)

        @pl.when(jnp.logical_and(cid == 1, sid == NS - 1))
        def _():
            seed(fRs_hbm, LAST)

        @pl.when(cid == 1)
        def _():
            pltpu.sync_copy(zw_hbm, accw.at[pl.ds(row0, RPT)])

        def issue_feat(j, b):
            @pl.when(cid == 0)
            def _():
                pltpu.async_copy(fL_hbm.at[src_v.at[j]], rows_v.at[b],
                                 fsem[b])

            @pl.when(cid == 1)
            def _():
                pltpu.async_copy(fR_hbm.at[src_v.at[j]], rows_v.at[b],
                                 fsem[b])

        ebase = sid * EPT

        def issue_ew(j, b):
            pltpu.async_copy(ew_hbm.at[pl.ds(ebase + j * B, B)],
                             ew_v.at[b], wsem[b])

        for b in range(NBUF):
            issue_feat(b, b)

            @pl.when(cid == 1)
            def _():
                issue_ew(b, b)

        plsc.subcore_barrier()

        def fbody(g, carry):
            for b in range(NBUF):
                j = g * NBUF + b
                pltpu.make_async_copy(fL_hbm.at[src_v.at[j]], rows_v.at[b],
                                      fsem[b]).wait()
                pltpu.sync_copy(rows_v.at[b], accf.at[dst_v.at[j]], add=True)

                @pl.when(j + NBUF < NB)
                def _():
                    issue_feat(j + NBUF, b)

                @pl.when(cid == 1)
                def _():
                    pltpu.make_async_copy(
                        ew_hbm.at[pl.ds(ebase + j * B, B)], ew_v.at[b],
                        wsem[b]).wait()
                    pltpu.sync_copy(ew_v.at[b], accw.at[dst_v.at[j]],
                                    add=True)

                    @pl.when(j + NBUF < NB)
                    def _():
                        issue_ew(j + NBUF, b)
            return carry

        lax.fori_loop(0, NB // NBUF, fbody, 0)
        plsc.subcore_barrier()

        def publish(rows):
            col0 = cid * DH
            pltpu.sync_copy(accf.at[pl.ds(row0, rows)],
                            out_hbm.at[pl.ds(row0, rows), pl.ds(col0, DH)])

            @pl.when(cid == 1)
            def _():
                pltpu.sync_copy(
                    accw.at[pl.ds(row0, rows)],
                    out_hbm.at[pl.ds(row0, rows), pl.ds(D, DE)])

        @pl.when(sid < NS - 1)
        def _():
            publish(RPT)

        @pl.when(sid == NS - 1)
        def _():
            publish(LAST)

    return k(featL, featR, featLs, featRs, src2, dst2, edge_w, zw)


def kernel(feat, edge_index, edge_w, eps):
    featL, featR, featLs, featRs = _split(feat, eps)
    src2 = edge_index[0].reshape(NS, NB, B)
    dst2 = edge_index[1].reshape(NS, NB, B)
    zw = jnp.zeros((RPT, DE), jnp.float32)
    return _sc_gin(featL, featR, featLs, featRs, src2, dst2, edge_w, zw)


# speedup vs baseline: 1.0963x; 1.0963x over previous
import functools

import jax
import jax.numpy as jnp
from jax import lax
from jax.experimental import pallas as pl
from jax.experimental.pallas import tpu as pltpu
from jax.experimental.pallas import tpu_sc as plsc

N = 10000
D = 128
DE = 16
E = 320000
DO = D + DE

NC = 2
NS = 16
DH = D // NC
EPT = E // NS
B = 80
NB = EPT // B
NP = 10240
RPT = NP // NS
LAST = N - (NS - 1) * RPT
NBUF = 5
VL = 16


def _scale(feat, eps):
    R = 2000

    def body(eps_ref, feat_ref, out_ref):
        out_ref[...] = (1.0 + eps_ref[0]) * feat_ref[...]

    return pl.pallas_call(
        body,
        grid=(N // R,),
        in_specs=[
            pl.BlockSpec(memory_space=pltpu.SMEM),
            pl.BlockSpec((R, D), lambda i: (i, 0)),
        ],
        out_specs=pl.BlockSpec((R, D), lambda i: (i, 0)),
        out_shape=jax.ShapeDtypeStruct((N, D), jnp.float32),
    )(eps, feat)


def _sc_gin(featD, featS, src2, dst2, edge_w, zw):
    mesh = plsc.VectorSubcoreMesh(
        core_axis_name="c", subcore_axis_name="s", num_cores=NC,
        num_subcores=NS)

    @functools.partial(
        pl.kernel,
        out_type=jax.ShapeDtypeStruct((N, DO), jnp.float32),
        mesh=mesh,
        compiler_params=pltpu.CompilerParams(use_tc_tiling_on_sc=False),
        scratch_types=[
            pltpu.VMEM((NB, B), jnp.int32),
            pltpu.VMEM((NB, B), jnp.int32),
            pltpu.VMEM((NBUF, B, DH), jnp.float32),
            pltpu.VMEM((NBUF, B, DE), jnp.float32),
            pltpu.VMEM_SHARED((NP, DH), jnp.float32),
            pltpu.VMEM_SHARED((NP, DE), jnp.float32),
        ] + [pltpu.SemaphoreType.DMA] * (2 * NBUF),
    )
    def k(fD_hbm, fS_hbm, src_hbm, dst_hbm, ew_hbm, zw_hbm, out_hbm,
          src_v, dst_v, rows_v, ew_v, accf, accw, *sems):
        fsem = sems[:NBUF]
        wsem = sems[NBUF:]
        cid = lax.axis_index("c")
        sid = lax.axis_index("s")
        row0 = sid * RPT
        col0 = cid * DH

        pltpu.sync_copy(src_hbm.at[sid], src_v)
        pltpu.sync_copy(dst_hbm.at[sid], dst_v)

        def xform(r, carry):
            for kk in range(B // VL):
                s = src_v[r, pl.ds(kk * VL, VL)]
                src_v[r, pl.ds(kk * VL, VL)] = s * 2 + cid
            return carry

        lax.fori_loop(0, NB, xform, 0)

        def seed(rows):
            pltpu.sync_copy(
                fS_hbm.at[pl.ds(row0, rows), pl.ds(col0, DH)],
                accf.at[pl.ds(row0, rows)])

        @pl.when(sid < NS - 1)
        def _():
            seed(RPT)

        @pl.when(sid == NS - 1)
        def _():
            seed(LAST)

        @pl.when(cid == 1)
        def _():
            pltpu.sync_copy(zw_hbm, accw.at[pl.ds(row0, RPT)])

        def issue_feat(j, b):
            pltpu.async_copy(fD_hbm.at[src_v.at[j]], rows_v.at[b], fsem[b])

        ebase = sid * EPT

        def issue_ew(j, b):
            pltpu.async_copy(ew_hbm.at[pl.ds(ebase + j * B, B)],
                             ew_v.at[b], wsem[b])

        for b in range(NBUF):
            issue_feat(b, b)

            @pl.when(cid == 1)
            def _():
                issue_ew(b, b)

        plsc.subcore_barrier()

        def fbody(g, carry):
            for b in range(NBUF):
                j = g * NBUF + b
                pltpu.make_async_copy(fD_hbm.at[src_v.at[j]], rows_v.at[b],
                                      fsem[b]).wait()
                pltpu.sync_copy(rows_v.at[b], accf.at[dst_v.at[j]], add=True)

                @pl.when(j + NBUF < NB)
                def _():
                    issue_feat(j + NBUF, b)

                @pl.when(cid == 1)
                def _():
                    pltpu.make_async_copy(
                        ew_hbm.at[pl.ds(ebase + j * B, B)], ew_v.at[b],
                        wsem[b]).wait()
                    pltpu.sync_copy(ew_v.at[b], accw.at[dst_v.at[j]],
                                    add=True)

                    @pl.when(j + NBUF < NB)
                    def _():
                        issue_ew(j + NBUF, b)
            return carry

        lax.fori_loop(0, NB // NBUF, fbody, 0)
        plsc.subcore_barrier()

        def publish(rows):
            pltpu.sync_copy(accf.at[pl.ds(row0, rows)],
                            out_hbm.at[pl.ds(row0, rows), pl.ds(col0, DH)])

            @pl.when(cid == 1)
            def _():
                pltpu.sync_copy(
                    accw.at[pl.ds(row0, rows)],
                    out_hbm.at[pl.ds(row0, rows), pl.ds(D, DE)])

        @pl.when(sid < NS - 1)
        def _():
            publish(RPT)

        @pl.when(sid == NS - 1)
        def _():
            publish(LAST)

    return k(featD, featS, src2, dst2, edge_w, zw)


def kernel(feat, edge_index, edge_w, eps):
    featD = feat.reshape(2 * N, DH)
    featS = _scale(feat, eps)
    src2 = edge_index[0].reshape(NS, NB, B)
    dst2 = edge_index[1].reshape(NS, NB, B)
    zw = jnp.zeros((RPT, DE), jnp.float32)
    return _sc_gin(featD, featS, src2, dst2, edge_w, zw)
